# baseline (speedup 1.0000x reference)
import functools
import math

import jax
import jax.numpy as jnp
from jax import lax
from jax.experimental import pallas as pl
from jax.experimental.pallas import tpu as pltpu
from jax.experimental.pallas import tpu_sc as plsc

D_LANES = 16

NUM_CORES = 2
NUM_SUBCORES = 16
NUM_WORKERS = NUM_CORES * NUM_SUBCORES


@functools.lru_cache(maxsize=None)
def _make_embed(B, V, D, chunk):
    assert B % (NUM_WORKERS * chunk) == 0
    b_per_w = B // NUM_WORKERS
    n_chunks = b_per_w // chunk
    scale = math.sqrt(D)
    vregs_per_row = D // D_LANES

    mesh = plsc.VectorSubcoreMesh(
        core_axis_name="c", subcore_axis_name="s")

    @functools.partial(
        pl.kernel,
        out_type=jax.ShapeDtypeStruct((B, D), jnp.float32),
        mesh=mesh,
        scratch_types=[
            pltpu.VMEM((chunk,), jnp.int32),
            pltpu.VMEM((chunk, D), jnp.float32),
            pltpu.SemaphoreType.DMA,
        ],
        compiler_params=pltpu.CompilerParams(use_tc_tiling_on_sc=False),
    )
    def embed(table_hbm, idx_hbm, out_hbm, idx_v, rows_v, sem):
        wid = lax.axis_index("s") * NUM_CORES + lax.axis_index("c")
        base = wid * b_per_w

        def chunk_body(g, carry):
            off = base + g * chunk
            pltpu.sync_copy(idx_hbm.at[pl.ds(off, chunk)], idx_v)
            pltpu.async_copy(table_hbm.at[idx_v], rows_v, sem).wait()

            def scale_body(r, c2):
                for j in range(vregs_per_row):
                    sl = pl.ds(j * D_LANES, D_LANES)
                    rows_v[r, sl] = rows_v[r, sl] * scale
                return c2

            lax.fori_loop(0, chunk, scale_body, 0)
            pltpu.sync_copy(rows_v, out_hbm.at[pl.ds(off, chunk)])
            return carry

        lax.fori_loop(0, n_chunks, chunk_body, 0)

    return embed


def kernel(x, table):
    V, D = table.shape
    B = x.size
    out = _make_embed(B, V, D, 512)(table, x.reshape(B))
    return out.reshape(*x.shape, D)


# speedup vs baseline: 1.1408x; 1.1408x over previous
import functools
import math

import jax
import jax.numpy as jnp
from jax import lax
from jax.experimental import pallas as pl
from jax.experimental.pallas import tpu as pltpu
from jax.experimental.pallas import tpu_sc as plsc

D_LANES = 16

NUM_CORES = 2
NUM_SUBCORES = 16
NUM_WORKERS = NUM_CORES * NUM_SUBCORES


@functools.lru_cache(maxsize=None)
def _make_embed(B, V, D, chunk):
    assert B % (NUM_WORKERS * chunk) == 0
    b_per_w = B // NUM_WORKERS
    n = b_per_w // chunk
    assert n % 2 == 0 and n >= 4
    scale = math.sqrt(D)
    vregs_per_row = D // D_LANES

    mesh = plsc.VectorSubcoreMesh(
        core_axis_name="c", subcore_axis_name="s")

    @functools.partial(
        pl.kernel,
        out_type=jax.ShapeDtypeStruct((B, D), jnp.float32),
        mesh=mesh,
        scratch_types=[
            pltpu.VMEM((b_per_w,), jnp.int32),
            pltpu.VMEM((chunk, D), jnp.float32),
            pltpu.VMEM((chunk, D), jnp.float32),
            pltpu.VMEM((chunk, D), jnp.float32),
            pltpu.VMEM((chunk, D), jnp.float32),
            pltpu.SemaphoreType.DMA,
            pltpu.SemaphoreType.DMA,
            pltpu.SemaphoreType.DMA,
            pltpu.SemaphoreType.DMA,
        ],
        compiler_params=pltpu.CompilerParams(use_tc_tiling_on_sc=False),
    )
    def embed(table_hbm, idx_hbm, out_hbm, idx_v, g0, g1, o0, o1,
              gs0, gs1, ss0, ss1):
        wid = lax.axis_index("s") * NUM_CORES + lax.axis_index("c")
        base = wid * b_per_w
        g_bufs = (g0, g1)
        o_bufs = (o0, o1)
        g_sems = (gs0, gs1)
        s_sems = (ss0, ss1)

        pltpu.sync_copy(idx_hbm.at[pl.ds(base, b_per_w)], idx_v)

        def start_gather(c, p):
            pltpu.async_copy(
                table_hbm.at[idx_v.at[pl.ds(c * chunk, chunk)]],
                g_bufs[p], g_sems[p])

        def wait_gather(p):
            pltpu.make_async_copy(table_hbm.at[idx_v.at[pl.ds(0, chunk)]],
                                  g_bufs[p], g_sems[p]).wait()

        def scale_chunk(p):
            src = g_bufs[p]
            dst = o_bufs[p]

            @plsc.parallel_loop(0, chunk, unroll=8)
            def _(r):
                for j in range(vregs_per_row):
                    sl = pl.ds(j * D_LANES, D_LANES)
                    dst[r, sl] = src[r, sl] * scale

        def start_scatter(c, p):
            pltpu.async_copy(
                o_bufs[p], out_hbm.at[pl.ds(base + c * chunk, chunk)],
                s_sems[p])

        def wait_scatter(p):
            pltpu.make_async_copy(
                o_bufs[p], out_hbm.at[pl.ds(0, chunk)], s_sems[p]).wait()

        start_gather(0, 0)
        start_gather(1, 1)
        wait_gather(0)
        scale_chunk(0)
        start_scatter(0, 0)
        start_gather(2, 0)
        wait_gather(1)
        scale_chunk(1)
        start_scatter(1, 1)
        start_gather(3, 1)

        def pair_body(t, carry):
            c = 2 * t
            for p in range(2):
                wait_gather(p)
                wait_scatter(p)
                scale_chunk(p)
                start_scatter(c + p, p)
                start_gather(c + p + 2, p)
            return carry

        lax.fori_loop(1, n // 2 - 1, pair_body, 0)

        for p in range(2):
            wait_gather(p)
            wait_scatter(p)
            scale_chunk(p)
            start_scatter(n - 2 + p, p)
        wait_scatter(0)
        wait_scatter(1)

    return embed


def kernel(x, table):
    V, D = table.shape
    B = x.size
    out = _make_embed(B, V, D, 400)(table, x.reshape(B))
    return out.reshape(*x.shape, D)
